# baseline (speedup 1.0000x reference)
import jax
import jax.numpy as jnp
from jax import lax
from jax.experimental import pallas as pl
from jax.experimental.pallas import tpu as pltpu
from jax.experimental.pallas import tpu_sc as plsc

P = 8
B, L, D, N = 4, 4096, 256, 1000
RPW = 128
NPADIN = 1024
WIN = 80
NG = D // 32

_INV = [1.0, 0.5, 1.0 / 3.0, 0.25, 0.2, 1.0 / 6.0, 1.0 / 7.0, 0.125]


def _inv_small_int(g):
  inv = jnp.float32(_INV[7])
  for k in range(6, -1, -1):
    inv = jnp.where(g == k + 1, jnp.float32(_INV[k]), inv)
  return inv


def _floor_nonneg(x):
  f = x.astype(jnp.int32)
  return f - (f.astype(jnp.float32) > x).astype(jnp.int32)


def _window_start(s):
  rs0 = jnp.maximum(s - 0.5, 0.0)
  w0 = (_floor_nonneg(rs0) // 8) * 8
  return jnp.minimum(w0, L - WIN)


def _decode_pair(w):
  lo = lax.bitcast_convert_type(w << 16, jnp.float32)
  hi = lax.bitcast_convert_type(w, jnp.float32)
  return lo, hi


def _body(starts_hbm, ends_hbm, feat_hbm, out_hbm, win_v, out_v,
          box_sp, s_sm, e_sm, sem0, sem1, osem0, osem1):
  cid = lax.axis_index("c")
  sid = lax.axis_index("s")
  wid = cid * 16 + sid
  b = wid // 8
  base = (wid % 8) * RPW

  pltpu.sync_copy(starts_hbm.at[b, pl.ds(base, RPW)], box_sp.at[sid, 0])
  pltpu.sync_copy(ends_hbm.at[b, pl.ds(base, RPW)], box_sp.at[sid, 1])
  pltpu.sync_copy(box_sp.at[sid, 0], s_sm)
  pltpu.sync_copy(box_sp.at[sid, 1], e_sm)

  sems = (sem0, sem1)
  osems = (osem0, osem1)

  def issue(j, buf):
    w0 = _window_start(s_sm[j])
    pltpu.make_async_copy(
        feat_hbm.at[b, pl.ds(w0, WIN)], win_v.at[buf], sems[buf]).start()

  def wait(buf):
    pltpu.make_async_copy(
        feat_hbm.at[b, pl.ds(0, WIN)], win_v.at[buf], sems[buf]).wait()

  def owait(jprev, buf):
    pred = jnp.logical_and(jprev >= 0, base + jprev < N)

    @pl.when(pred)
    def _():
      pltpu.make_async_copy(
          out_v.at[buf], out_hbm.at[b, base], osems[buf]).wait()

  def compute(j, buf):
    s = s_sm[j]
    e = e_sm[j]
    roi_start = s - 0.5
    roi_h = e - s
    bin_h = roi_h * jnp.float32(1.0 / P)
    gi = bin_h.astype(jnp.int32)
    g0 = gi + (gi.astype(jnp.float32) < bin_h).astype(jnp.int32)
    g0 = jnp.maximum(g0, 1)
    inv_cnt = _inv_small_int(g0)
    step = bin_h * inv_cnt
    w0 = _window_start(s)

    for ph in range(P):
      y0 = roi_start + jnp.float32(ph) * bin_h + jnp.float32(0.5) * step

      z = tuple(jnp.zeros((16,), jnp.float32) for _ in range(NG))

      @plsc.parallel_loop(0, g0, unroll=2, carry=(z, z))
      def accs(iy, accs):
        y = y0 + iy.astype(jnp.float32) * step
        yc = jnp.minimum(jnp.maximum(y, 0.0), jnp.float32(L - 1))
        ylow = _floor_nonneg(yc)
        ly = yc - ylow.astype(jnp.float32)
        hy = 1.0 - ly
        rel = ylow - w0
        relh = jnp.minimum(rel + 1, WIN - 1)
        ae, ao = accs
        ne, no = [], []
        for c in range(NG):
          vl_lo, vl_hi = _decode_pair(win_v[buf, rel, pl.ds(c * 16, 16)])
          vh_lo, vh_hi = _decode_pair(win_v[buf, relh, pl.ds(c * 16, 16)])
          ne.append(ae[c] + hy * vl_lo + ly * vh_lo)
          no.append(ao[c] + hy * vl_hi + ly * vh_hi)
        return tuple(ne), tuple(no)

      ae, ao = accs
      for c in range(NG):
        out_v[buf, ph, pl.ds(c * 32, 16)] = ae[c] * inv_cnt
        out_v[buf, ph, pl.ds(c * 32 + 16, 16)] = ao[c] * inv_cnt

    @pl.when(base + j < N)
    def _():
      pltpu.make_async_copy(
          out_v.at[buf], out_hbm.at[b, base + j], osems[buf]).start()

  issue(jnp.int32(0), 0)

  def pair(t, carry):
    j0 = 2 * t
    issue(j0 + 1, 1)
    wait(0)
    owait(j0 - 2, 0)
    compute(j0, 0)

    @pl.when(j0 + 2 < RPW)
    def _():
      issue(j0 + 2, 0)

    wait(1)
    owait(j0 - 1, 1)
    compute(j0 + 1, 1)
    return carry

  lax.fori_loop(0, RPW // 2, pair, 0)
  owait(RPW - 2, 0)
  owait(RPW - 1, 1)


@jax.jit
def kernel(feat, roi_boxxes_batch):
  starts = roi_boxxes_batch[..., 0]
  ends = roi_boxxes_batch[..., 1]
  pad = NPADIN - N
  starts = jnp.pad(starts, ((0, 0), (0, pad)))
  ends = jnp.pad(ends, ((0, 0), (0, pad)), constant_values=8.0)
  fb = feat.astype(jnp.bfloat16).reshape(B, L, D // 32, 2, 16)
  feat_w = lax.bitcast_convert_type(
      jnp.swapaxes(fb, -1, -2), jnp.int32).reshape(B, L, D // 2)

  mesh = plsc.VectorSubcoreMesh(
      core_axis_name="c", subcore_axis_name="s", num_cores=2, num_subcores=16)
  run = pl.kernel(
      _body,
      out_type=jax.ShapeDtypeStruct((B, N, P, D), jnp.float32),
      mesh=mesh,
      scratch_types=[
          pltpu.VMEM((2, WIN, D // 2), jnp.int32),
          pltpu.VMEM((2, P, D), jnp.float32),
          pltpu.VMEM_SHARED((16, 2, RPW), jnp.float32),
          pltpu.SMEM((RPW,), jnp.float32),
          pltpu.SMEM((RPW,), jnp.float32),
          pltpu.SemaphoreType.DMA,
          pltpu.SemaphoreType.DMA,
          pltpu.SemaphoreType.DMA,
          pltpu.SemaphoreType.DMA,
      ],
  )
  return run(starts, ends, feat_w)


# speedup vs baseline: 1.3720x; 1.3720x over previous
import jax
import jax.numpy as jnp
from jax import lax
from jax.experimental import pallas as pl
from jax.experimental.pallas import tpu as pltpu
from jax.experimental.pallas import tpu_sc as plsc

P = 8
B, L, D, N = 4, 4096, 256, 1000
RPW = 128
NPADIN = 1024
WIN = 80
NG = D // 32

_INV = [1.0, 0.5, 1.0 / 3.0, 0.25, 0.2, 1.0 / 6.0, 1.0 / 7.0, 0.125]


def _inv_small_int(g):
  inv = jnp.float32(_INV[7])
  for k in range(6, -1, -1):
    inv = jnp.where(g == k + 1, jnp.float32(_INV[k]), inv)
  return inv


def _floor_nonneg(x):
  f = x.astype(jnp.int32)
  return f - (f.astype(jnp.float32) > x).astype(jnp.int32)


def _window_start(s):
  rs0 = jnp.maximum(s - 0.5, 0.0)
  w0 = (_floor_nonneg(rs0) // 8) * 8
  return jnp.minimum(w0, L - WIN)


def _decode_pair(w):
  lo = lax.bitcast_convert_type(w << 16, jnp.float32)
  hi = lax.bitcast_convert_type(w & -65536, jnp.float32)
  return lo, hi


def _body(starts_hbm, ends_hbm, feat_hbm, out_hbm, win_v, out_v,
          box_sp, s_sm, e_sm, sem0, sem1, osem0, osem1):
  cid = lax.axis_index("c")
  sid = lax.axis_index("s")
  wid = cid * 16 + sid
  b = wid // 8
  base = (wid % 8) * RPW

  pltpu.sync_copy(starts_hbm.at[b, pl.ds(base, RPW)], box_sp.at[sid, 0])
  pltpu.sync_copy(ends_hbm.at[b, pl.ds(base, RPW)], box_sp.at[sid, 1])
  pltpu.sync_copy(box_sp.at[sid, 0], s_sm)
  pltpu.sync_copy(box_sp.at[sid, 1], e_sm)

  sems = (sem0, sem1)
  osems = (osem0, osem1)

  def issue(j, buf):
    w0 = _window_start(s_sm[j])
    pltpu.make_async_copy(
        feat_hbm.at[b, pl.ds(w0, WIN)], win_v.at[buf], sems[buf]).start()

  def wait(buf):
    pltpu.make_async_copy(
        feat_hbm.at[b, pl.ds(0, WIN)], win_v.at[buf], sems[buf]).wait()

  def owait(jprev, buf):
    pred = jnp.logical_and(jprev >= 0, base + jprev < N)

    @pl.when(pred)
    def _():
      pltpu.make_async_copy(
          out_v.at[buf], out_hbm.at[b, base], osems[buf]).wait()

  def compute(j, buf):
    s = s_sm[j]
    e = e_sm[j]
    roi_start = s - 0.5
    roi_h = e - s
    bin_h = roi_h * jnp.float32(1.0 / P)
    gi = bin_h.astype(jnp.int32)
    g0 = gi + (gi.astype(jnp.float32) < bin_h).astype(jnp.int32)
    g0 = jnp.maximum(g0, 1)
    inv_cnt = _inv_small_int(g0)
    step = bin_h * inv_cnt
    w0 = _window_start(s)

    for ph in range(P):
      y0 = roi_start + jnp.float32(ph) * bin_h + jnp.float32(0.5) * step

      z = tuple(jnp.zeros((16,), jnp.float32) for _ in range(NG))

      @plsc.parallel_loop(0, g0, unroll=2, carry=(z, z))
      def accs(iy, accs):
        y = y0 + iy.astype(jnp.float32) * step
        yc = jnp.minimum(jnp.maximum(y, 0.0), jnp.float32(L - 1))
        ylow = _floor_nonneg(yc)
        ly = yc - ylow.astype(jnp.float32)
        hy = 1.0 - ly
        rel = ylow - w0
        relh = jnp.minimum(rel + 1, WIN - 1)
        ae, ao = accs
        ne, no = [], []
        for c in range(NG):
          vl_lo, vl_hi = _decode_pair(win_v[buf, rel, pl.ds(c * 16, 16)])
          vh_lo, vh_hi = _decode_pair(win_v[buf, relh, pl.ds(c * 16, 16)])
          ne.append(ae[c] + hy * vl_lo + ly * vh_lo)
          no.append(ao[c] + hy * vl_hi + ly * vh_hi)
        return tuple(ne), tuple(no)

      ae, ao = accs
      for c in range(NG):
        out_v[buf, ph, pl.ds(c * 32, 16)] = ae[c] * inv_cnt
        out_v[buf, ph, pl.ds(c * 32 + 16, 16)] = ao[c] * inv_cnt

    @pl.when(base + j < N)
    def _():
      pltpu.make_async_copy(
          out_v.at[buf], out_hbm.at[b, base + j], osems[buf]).start()

  issue(jnp.int32(0), 0)

  def pair(t, carry):
    j0 = 2 * t
    issue(j0 + 1, 1)
    wait(0)
    owait(j0 - 2, 0)
    compute(j0, 0)

    @pl.when(j0 + 2 < RPW)
    def _():
      issue(j0 + 2, 0)

    wait(1)
    owait(j0 - 1, 1)
    compute(j0 + 1, 1)
    return carry

  lax.fori_loop(0, RPW // 2, pair, 0)
  owait(RPW - 2, 0)
  owait(RPW - 1, 1)


@jax.jit
def kernel(feat, roi_boxxes_batch):
  starts = roi_boxxes_batch[..., 0]
  ends = roi_boxxes_batch[..., 1]
  pad = NPADIN - N
  starts = jnp.pad(starts, ((0, 0), (0, pad)))
  ends = jnp.pad(ends, ((0, 0), (0, pad)), constant_values=8.0)
  fb = feat.astype(jnp.bfloat16).reshape(B, L, D // 32, 2, 16)
  feat_w = lax.bitcast_convert_type(
      jnp.swapaxes(fb, -1, -2), jnp.int32).reshape(B, L, D // 2)

  mesh = plsc.VectorSubcoreMesh(
      core_axis_name="c", subcore_axis_name="s", num_cores=2, num_subcores=16)
  run = pl.kernel(
      _body,
      out_type=jax.ShapeDtypeStruct((B, N, P, D), jnp.float32),
      mesh=mesh,
      scratch_types=[
          pltpu.VMEM((2, WIN, D // 2), jnp.int32),
          pltpu.VMEM((2, P, D), jnp.float32),
          pltpu.VMEM_SHARED((16, 2, RPW), jnp.float32),
          pltpu.SMEM((RPW,), jnp.float32),
          pltpu.SMEM((RPW,), jnp.float32),
          pltpu.SemaphoreType.DMA,
          pltpu.SemaphoreType.DMA,
          pltpu.SemaphoreType.DMA,
          pltpu.SemaphoreType.DMA,
      ],
  )
  return run(starts, ends, feat_w)
